# baseline (speedup 1.0000x reference)
import dataclasses
import functools

import jax
import jax.numpy as jnp
import numpy as np
from jax import lax
from jax.experimental import pallas as pl
from jax.experimental.pallas import tpu as pltpu
from jax.experimental.pallas import tpu_sc as plsc

N = 10000
E = 160000
C = 256
ALPHA = 0.1
THETA = 0.5
LAYER = 2
BETA = float(np.log(THETA / LAYER + 1.0))
LN_EPS = 1e-5

NC = 2
NS = 16
LANES = 16

CHUNK = 64
E_PAD = 163840
PAD_DST = N
N_PAD = 10240
EPT_B = E_PAD // NS
NCHUNK = EPT_B // CHUNK
EPT_A = E_PAD // (NC * NS)
STRIPE = N_PAD // NS
ROWS_T = N // NS

HALF = C // 2

_mesh = plsc.VectorSubcoreMesh(core_axis_name="c", subcore_axis_name="s")

_sc_params = pltpu.CompilerParams()
if "needs_layout_passes" in pltpu.CompilerParams.__dataclass_fields__:
    _sc_params = dataclasses.replace(_sc_params, needs_layout_passes=False)


@functools.partial(
    pl.kernel,
    out_type=jax.ShapeDtypeStruct((NC, N_PAD), jnp.float32),
    mesh=_mesh,
    compiler_params=_sc_params,
    scratch_types=[
        pltpu.VMEM((EPT_A,), jnp.int32),
        pltpu.VMEM((N_PAD,), jnp.float32),
        pltpu.VMEM((NS, STRIPE), jnp.float32),
        pltpu.VMEM((STRIPE,), jnp.float32),
        pltpu.VMEM_SHARED((NS, N_PAD), jnp.float32),
    ],
)
def _deg_kernel(dst_hbm, deg_out, dst_v, hist_v, red_v, out_v, shared):
    c = lax.axis_index("c")
    s = lax.axis_index("s")
    w = c * NS + s

    zeros16 = jnp.zeros((LANES,), jnp.float32)
    ones16 = jnp.ones((LANES,), jnp.float32)

    @pl.loop(0, N_PAD // LANES)
    def _(i):
        hist_v[pl.ds(i * LANES, LANES)] = zeros16

    pltpu.sync_copy(dst_hbm.at[w], dst_v)

    @pl.loop(0, EPT_A // LANES)
    def _(i):
        idx = dst_v[pl.ds(i * LANES, LANES)]
        plsc.addupdate_scatter(hist_v, [idx], ones16)

    pltpu.sync_copy(hist_v, shared.at[s])
    plsc.subcore_barrier()

    pltpu.sync_copy(shared.at[:, pl.ds(s * STRIPE, STRIPE)], red_v)

    @pl.loop(0, STRIPE // LANES)
    def _(k):
        acc = red_v[0, pl.ds(k * LANES, LANES)]
        for r in range(1, NS):
            acc = acc + red_v[r, pl.ds(k * LANES, LANES)]
        out_v[pl.ds(k * LANES, LANES)] = acc

    pltpu.sync_copy(out_v, deg_out.at[c, pl.ds(s * STRIPE, STRIPE)])


NBUF = 4


@functools.partial(
    pl.kernel,
    out_type=jax.ShapeDtypeStruct((NC, N_PAD, HALF), jnp.float32),
    mesh=_mesh,
    compiler_params=_sc_params,
    scratch_types=[
        pltpu.VMEM((EPT_B,), jnp.int32),
        pltpu.VMEM((CHUNK,), jnp.int32),
        pltpu.VMEM((CHUNK,), jnp.int32),
        pltpu.VMEM((CHUNK,), jnp.int32),
        pltpu.VMEM((CHUNK,), jnp.int32),
        pltpu.VMEM((CHUNK,), jnp.int32),
        pltpu.VMEM((CHUNK,), jnp.int32),
        pltpu.VMEM((CHUNK,), jnp.int32),
        pltpu.VMEM((CHUNK,), jnp.int32),
        pltpu.VMEM((NBUF, CHUNK, HALF), jnp.float32),
        pltpu.VMEM_SHARED((N_PAD, HALF), jnp.float32),
        pltpu.SemaphoreType.DMA,
        pltpu.SemaphoreType.DMA,
        pltpu.SemaphoreType.DMA,
        pltpu.SemaphoreType.DMA,
        pltpu.SemaphoreType.DMA,
        pltpu.SemaphoreType.DMA,
        pltpu.SemaphoreType.DMA,
        pltpu.SemaphoreType.DMA,
    ],
)
def _segsum_kernel(g_hbm, pk_hbm, z_hbm, s_out,
                   pk_v, si0, si1, si2, si3, di0, di1, di2, di3,
                   rows_v, acc,
                   gs0, gs1, gs2, gs3, ss0, ss1, ss2, ss3):
    c = lax.axis_index("c")
    s = lax.axis_index("s")
    srcs = [si0, si1, si2, si3]
    dsts = [di0, di1, di2, di3]
    gsems = [gs0, gs1, gs2, gs3]
    ssems = [ss0, ss1, ss2, ss3]
    base = c * N

    pltpu.sync_copy(z_hbm, acc.at[pl.ds(s * STRIPE, STRIPE)])
    pltpu.sync_copy(pk_hbm.at[s], pk_v)
    plsc.subcore_barrier()

    def unpack(j, b):
        @pl.loop(0, CHUNK // LANES)
        def _(k):
            w = pk_v[pl.ds(j * CHUNK + k * LANES, LANES)]
            srcs[b][pl.ds(k * LANES, LANES)] = (w & 0xFFFF) + base
            dsts[b][pl.ds(k * LANES, LANES)] = lax.shift_right_logical(w, 16)

    def wait_scatter(b):
        pltpu.make_async_copy(rows_v.at[b], acc.at[dsts[b]], ssems[b]).wait()

    for b in range(2):
        unpack(b, b)
        pltpu.async_copy(g_hbm.at[srcs[b]], rows_v.at[b], gsems[b])

    @pl.loop(0, NCHUNK, step=NBUF)
    def _(j0):
        for b in range(NBUF):
            j = j0 + b
            pltpu.make_async_copy(g_hbm.at[srcs[b]], rows_v.at[b],
                                  gsems[b]).wait()
            pltpu.async_copy(rows_v.at[b], acc.at[dsts[b]], ssems[b],
                             add=True)
            b2 = (b + 2) % NBUF
            nxt = j + 2

            @pl.when(nxt < NCHUNK)
            def _():
                @pl.when(j >= 2)
                def _():
                    wait_scatter(b2)

                unpack(nxt, b2)
                pltpu.async_copy(g_hbm.at[srcs[b2]], rows_v.at[b2], gsems[b2])

    for b in range(NBUF):
        wait_scatter(b)

    plsc.subcore_barrier()
    pltpu.sync_copy(
        acc.at[pl.ds(s * STRIPE, STRIPE)],
        s_out.at[c, pl.ds(s * STRIPE, STRIPE)],
    )


def _prep_body(x_ref, g0_ref, b0_ref, d0_ref, d1_ref, h_ref, g_ref, dinv_ref):
    xv = x_ref[...]
    mean = jnp.mean(xv, axis=-1, keepdims=True)
    var = jnp.mean(jnp.square(xv - mean), axis=-1, keepdims=True)
    hv = (xv - mean) * lax.rsqrt(var + LN_EPS) * g0_ref[...] + b0_ref[...]
    hv = jnp.maximum(hv, 0.0)
    deg = d0_ref[...] + d1_ref[...] + 1.0
    dinv = lax.rsqrt(deg)
    gv = dinv * hv
    h_ref[...] = hv
    dinv_ref[...] = dinv
    g_ref[...] = jnp.stack([gv[:, :HALF], gv[:, HALF:]], axis=0)


def _combine_body(s_ref, h_ref, dinv_ref, w_ref, out_ref):
    s_all = jnp.concatenate([s_ref[0], s_ref[1]], axis=-1)
    dinv = dinv_ref[...]
    agg = dinv * s_all + (dinv * dinv) * h_ref[...]
    t = (1.0 - ALPHA) * agg + ALPHA * h_ref[...]
    mm = lax.dot(
        t, w_ref[...],
        precision=lax.Precision.HIGHEST,
        preferred_element_type=jnp.float32,
    )
    out_ref[...] = (1.0 - BETA) * t + BETA * mm


_ROWS_BLK = 1000
_GRID = N // _ROWS_BLK


def kernel(x, edge_index, ln_gamma, ln_beta, weight1):
    src = edge_index[0]
    dst = edge_index[1]
    npad = E_PAD - E
    src_p = jnp.concatenate([src, jnp.zeros((npad,), jnp.int32)])
    dst_p = jnp.concatenate([dst, jnp.full((npad,), PAD_DST, jnp.int32)])

    dstA = dst_p.reshape(NC * NS, EPT_A)
    pk = ((dst_p << 16) | src_p).reshape(NS, EPT_B)

    degA = _deg_kernel(dstA)
    d0 = degA[0, :N].reshape(N, 1)
    d1 = degA[1, :N].reshape(N, 1)

    h, g_st, dinv = pl.pallas_call(
        _prep_body,
        grid=(_GRID,),
        in_specs=[
            pl.BlockSpec((_ROWS_BLK, C), lambda i: (i, 0)),
            pl.BlockSpec((1, C), lambda i: (0, 0)),
            pl.BlockSpec((1, C), lambda i: (0, 0)),
            pl.BlockSpec((_ROWS_BLK, 1), lambda i: (i, 0)),
            pl.BlockSpec((_ROWS_BLK, 1), lambda i: (i, 0)),
        ],
        out_specs=[
            pl.BlockSpec((_ROWS_BLK, C), lambda i: (i, 0)),
            pl.BlockSpec((NC, _ROWS_BLK, HALF), lambda i: (0, i, 0)),
            pl.BlockSpec((_ROWS_BLK, 1), lambda i: (i, 0)),
        ],
        out_shape=[
            jax.ShapeDtypeStruct((N, C), jnp.float32),
            jax.ShapeDtypeStruct((NC, N, HALF), jnp.float32),
            jax.ShapeDtypeStruct((N, 1), jnp.float32),
        ],
    )(x, ln_gamma.reshape(1, C), ln_beta.reshape(1, C), d0, d1)

    g_flat = g_st.reshape(NC * N, HALF)
    zrows = jnp.zeros((STRIPE, HALF), jnp.float32)
    s_sum = _segsum_kernel(g_flat, pk, zrows)

    out = pl.pallas_call(
        _combine_body,
        grid=(_GRID,),
        in_specs=[
            pl.BlockSpec((NC, _ROWS_BLK, HALF), lambda i: (0, i, 0)),
            pl.BlockSpec((_ROWS_BLK, C), lambda i: (i, 0)),
            pl.BlockSpec((_ROWS_BLK, 1), lambda i: (i, 0)),
            pl.BlockSpec((C, C), lambda i: (0, 0)),
        ],
        out_specs=pl.BlockSpec((_ROWS_BLK, C), lambda i: (i, 0)),
        out_shape=jax.ShapeDtypeStruct((N, C), jnp.float32),
    )(s_sum, h, dinv, weight1)
    return out


# speedup vs baseline: 1.0189x; 1.0189x over previous
import dataclasses
import functools

import jax
import jax.numpy as jnp
import numpy as np
from jax import lax
from jax.experimental import pallas as pl
from jax.experimental.pallas import tpu as pltpu
from jax.experimental.pallas import tpu_sc as plsc

N = 10000
E = 160000
C = 256
ALPHA = 0.1
THETA = 0.5
LAYER = 2
BETA = float(np.log(THETA / LAYER + 1.0))
LN_EPS = 1e-5

NC = 2
NS = 16
LANES = 16

CHUNK = 128
E_PAD = 163840
PAD_DST = N
N_PAD = 10240
EPT_B = E_PAD // NS
NCHUNK = EPT_B // CHUNK
EPT_A = E_PAD // (NC * NS)
STRIPE = N_PAD // NS
ROWS_T = N // NS

HALF = C // 2
QCH = C // 4

_mesh = plsc.VectorSubcoreMesh(core_axis_name="c", subcore_axis_name="s")

_sc_params = pltpu.CompilerParams()
if "needs_layout_passes" in pltpu.CompilerParams.__dataclass_fields__:
    _sc_params = dataclasses.replace(_sc_params, needs_layout_passes=False)


@functools.partial(
    pl.kernel,
    out_type=jax.ShapeDtypeStruct((NC, N_PAD), jnp.float32),
    mesh=_mesh,
    compiler_params=_sc_params,
    scratch_types=[
        pltpu.VMEM((EPT_A,), jnp.int32),
        pltpu.VMEM((N_PAD,), jnp.float32),
        pltpu.VMEM((NS, STRIPE), jnp.float32),
        pltpu.VMEM((STRIPE,), jnp.float32),
        pltpu.VMEM_SHARED((NS, N_PAD), jnp.float32),
    ],
)
def _deg_kernel(dst_hbm, deg_out, dst_v, hist_v, red_v, out_v, shared):
    c = lax.axis_index("c")
    s = lax.axis_index("s")
    w = c * NS + s

    zeros16 = jnp.zeros((LANES,), jnp.float32)
    ones16 = jnp.ones((LANES,), jnp.float32)

    @pl.loop(0, N_PAD // LANES)
    def _(i):
        hist_v[pl.ds(i * LANES, LANES)] = zeros16

    pltpu.sync_copy(dst_hbm.at[w], dst_v)

    @pl.loop(0, EPT_A // LANES)
    def _(i):
        idx = dst_v[pl.ds(i * LANES, LANES)]
        plsc.addupdate_scatter(hist_v, [idx], ones16)

    pltpu.sync_copy(hist_v, shared.at[s])
    plsc.subcore_barrier()

    pltpu.sync_copy(shared.at[:, pl.ds(s * STRIPE, STRIPE)], red_v)

    @pl.loop(0, STRIPE // LANES)
    def _(k):
        acc = red_v[0, pl.ds(k * LANES, LANES)]
        for r in range(1, NS):
            acc = acc + red_v[r, pl.ds(k * LANES, LANES)]
        out_v[pl.ds(k * LANES, LANES)] = acc

    pltpu.sync_copy(out_v, deg_out.at[c, pl.ds(s * STRIPE, STRIPE)])


NBUF = 2


@functools.partial(
    pl.kernel,
    out_type=jax.ShapeDtypeStruct((NC, N_PAD, HALF), jnp.float32),
    mesh=_mesh,
    compiler_params=_sc_params,
    scratch_types=[
        pltpu.VMEM((EPT_B,), jnp.int32),
        pltpu.VMEM((CHUNK,), jnp.int32),
        pltpu.VMEM((CHUNK,), jnp.int32),
        pltpu.VMEM((CHUNK,), jnp.int32),
        pltpu.VMEM((CHUNK,), jnp.int32),
        pltpu.VMEM((NBUF, CHUNK, HALF), jnp.float32),
        pltpu.VMEM_SHARED((N_PAD, HALF), jnp.float32),
        pltpu.SemaphoreType.DMA,
        pltpu.SemaphoreType.DMA,
    ],
)
def _segsum_kernel(g_hbm, pk_hbm, z_hbm, s_out,
                   pk_v, si0, si1, di0, di1, rows_v, acc, gs0, gs1):
    c = lax.axis_index("c")
    s = lax.axis_index("s")
    srcs = [si0, si1]
    dsts = [di0, di1]
    gsems = [gs0, gs1]
    stripe = pl.ds(s * STRIPE, STRIPE)
    base = c * N

    pltpu.sync_copy(z_hbm, acc.at[stripe])
    pltpu.sync_copy(pk_hbm.at[s], pk_v)
    plsc.subcore_barrier()

    def unpack(j, b):
        @pl.loop(0, CHUNK // LANES)
        def _(k):
            w = pk_v[pl.ds(j * CHUNK + k * LANES, LANES)]
            srcs[b][pl.ds(k * LANES, LANES)] = (w & 0xFFFF) + base
            dsts[b][pl.ds(k * LANES, LANES)] = lax.shift_right_logical(w, 16)

    for b in range(NBUF):
        unpack(b, b)
        pltpu.async_copy(g_hbm.at[srcs[b]], rows_v.at[b], gsems[b])

    @pl.loop(0, NCHUNK, step=NBUF)
    def _(j0):
        for b in range(NBUF):
            j = j0 + b
            pltpu.make_async_copy(g_hbm.at[srcs[b]], rows_v.at[b],
                                  gsems[b]).wait()
            pltpu.sync_copy(rows_v.at[b], acc.at[dsts[b]], add=True)
            nxt = j + NBUF

            @pl.when(nxt < NCHUNK)
            def _():
                unpack(nxt, b)
                pltpu.async_copy(g_hbm.at[srcs[b]], rows_v.at[b], gsems[b])

    plsc.subcore_barrier()
    pltpu.sync_copy(acc.at[stripe], s_out.at[c, stripe])


def _prep_body(x_ref, g0_ref, b0_ref, d0_ref, d1_ref, h_ref, g_ref, dinv_ref):
    xv = x_ref[...]
    mean = jnp.mean(xv, axis=-1, keepdims=True)
    var = jnp.mean(jnp.square(xv - mean), axis=-1, keepdims=True)
    hv = (xv - mean) * lax.rsqrt(var + LN_EPS) * g0_ref[...] + b0_ref[...]
    hv = jnp.maximum(hv, 0.0)
    deg = d0_ref[...] + d1_ref[...] + 1.0
    dinv = lax.rsqrt(deg)
    gv = dinv * hv
    h_ref[...] = hv
    dinv_ref[...] = dinv
    g_ref[...] = jnp.stack([gv[:, :HALF], gv[:, HALF:]], axis=0)


def _combine_body(s_ref, h_ref, dinv_ref, w_ref, out_ref):
    s_all = jnp.concatenate([s_ref[0], s_ref[1]], axis=-1)
    dinv = dinv_ref[...]
    agg = dinv * s_all + (dinv * dinv) * h_ref[...]
    t = (1.0 - ALPHA) * agg + ALPHA * h_ref[...]
    mm = lax.dot(
        t, w_ref[...],
        precision=lax.Precision.HIGHEST,
        preferred_element_type=jnp.float32,
    )
    out_ref[...] = (1.0 - BETA) * t + BETA * mm


_ROWS_BLK = 1000
_GRID = N // _ROWS_BLK


def kernel(x, edge_index, ln_gamma, ln_beta, weight1):
    src = edge_index[0]
    dst = edge_index[1]
    npad = E_PAD - E
    src_p = jnp.concatenate([src, jnp.zeros((npad,), jnp.int32)])
    dst_p = jnp.concatenate([dst, jnp.full((npad,), PAD_DST, jnp.int32)])

    dstA = dst_p.reshape(NC * NS, EPT_A)
    pk = ((dst_p << 16) | src_p).reshape(NS, EPT_B)

    degA = _deg_kernel(dstA)
    d0 = degA[0, :N].reshape(N, 1)
    d1 = degA[1, :N].reshape(N, 1)

    h, g_st, dinv = pl.pallas_call(
        _prep_body,
        grid=(_GRID,),
        in_specs=[
            pl.BlockSpec((_ROWS_BLK, C), lambda i: (i, 0)),
            pl.BlockSpec((1, C), lambda i: (0, 0)),
            pl.BlockSpec((1, C), lambda i: (0, 0)),
            pl.BlockSpec((_ROWS_BLK, 1), lambda i: (i, 0)),
            pl.BlockSpec((_ROWS_BLK, 1), lambda i: (i, 0)),
        ],
        out_specs=[
            pl.BlockSpec((_ROWS_BLK, C), lambda i: (i, 0)),
            pl.BlockSpec((NC, _ROWS_BLK, HALF), lambda i: (0, i, 0)),
            pl.BlockSpec((_ROWS_BLK, 1), lambda i: (i, 0)),
        ],
        out_shape=[
            jax.ShapeDtypeStruct((N, C), jnp.float32),
            jax.ShapeDtypeStruct((NC, N, HALF), jnp.float32),
            jax.ShapeDtypeStruct((N, 1), jnp.float32),
        ],
    )(x, ln_gamma.reshape(1, C), ln_beta.reshape(1, C), d0, d1)

    g_flat = g_st.reshape(NC * N, HALF)
    zrows = jnp.zeros((STRIPE, HALF), jnp.float32)
    s_sum = _segsum_kernel(g_flat, pk, zrows)

    out = pl.pallas_call(
        _combine_body,
        grid=(_GRID,),
        in_specs=[
            pl.BlockSpec((NC, _ROWS_BLK, HALF), lambda i: (0, i, 0)),
            pl.BlockSpec((_ROWS_BLK, C), lambda i: (i, 0)),
            pl.BlockSpec((_ROWS_BLK, 1), lambda i: (i, 0)),
            pl.BlockSpec((C, C), lambda i: (0, 0)),
        ],
        out_specs=pl.BlockSpec((_ROWS_BLK, C), lambda i: (i, 0)),
        out_shape=jax.ShapeDtypeStruct((N, C), jnp.float32),
    )(s_sum, h, dinv, weight1)
    return out


# speedup vs baseline: 2.1924x; 2.1517x over previous
import dataclasses
import functools

import jax
import jax.numpy as jnp
import numpy as np
from jax import lax
from jax.experimental import pallas as pl
from jax.experimental.pallas import tpu as pltpu
from jax.experimental.pallas import tpu_sc as plsc

N = 10000
E = 160000
C = 256
ALPHA = 0.1
THETA = 0.5
LAYER = 2
BETA = float(np.log(THETA / LAYER + 1.0))
LN_EPS = 1e-5

NC = 2
NS = 16
LANES = 16

CHUNK = 80
E_PAD = 163840
PAD_DST = N
N_PAD = 10240
EPT_B = E // NS
NCHUNK = EPT_B // CHUNK
EPT_A = E_PAD // (NC * NS)
STRIPE = N_PAD // NS

HALF = C // 2
QCH = C // 4

_mesh = plsc.VectorSubcoreMesh(core_axis_name="c", subcore_axis_name="s")

_sc_params = pltpu.CompilerParams()
if "needs_layout_passes" in pltpu.CompilerParams.__dataclass_fields__:
    _sc_params = dataclasses.replace(_sc_params, needs_layout_passes=False)


@functools.partial(
    pl.kernel,
    out_type=jax.ShapeDtypeStruct((NC, N_PAD), jnp.float32),
    mesh=_mesh,
    compiler_params=_sc_params,
    scratch_types=[
        pltpu.VMEM((EPT_A,), jnp.int32),
        pltpu.VMEM((N_PAD,), jnp.float32),
        pltpu.VMEM((NS, STRIPE), jnp.float32),
        pltpu.VMEM((STRIPE,), jnp.float32),
        pltpu.VMEM_SHARED((NS, N_PAD), jnp.float32),
    ],
)
def _deg_kernel(dst_hbm, deg_out, dst_v, hist_v, red_v, out_v, shared):
    c = lax.axis_index("c")
    s = lax.axis_index("s")
    w = c * NS + s

    zeros16 = jnp.zeros((LANES,), jnp.float32)
    ones16 = jnp.ones((LANES,), jnp.float32)

    @pl.loop(0, N_PAD // LANES)
    def _(i):
        hist_v[pl.ds(i * LANES, LANES)] = zeros16

    pltpu.sync_copy(dst_hbm.at[w], dst_v)

    @pl.loop(0, EPT_A // LANES)
    def _(i):
        idx = dst_v[pl.ds(i * LANES, LANES)]
        plsc.addupdate_scatter(hist_v, [idx], ones16)

    pltpu.sync_copy(hist_v, shared.at[s])
    plsc.subcore_barrier()

    pltpu.sync_copy(shared.at[:, pl.ds(s * STRIPE, STRIPE)], red_v)

    @pl.loop(0, STRIPE // LANES)
    def _(k):
        acc = red_v[0, pl.ds(k * LANES, LANES)]
        for r in range(1, NS):
            acc = acc + red_v[r, pl.ds(k * LANES, LANES)]
        out_v[pl.ds(k * LANES, LANES)] = acc

    pltpu.sync_copy(out_v, deg_out.at[c, pl.ds(s * STRIPE, STRIPE)])


NBUF = 3
_NITER = ((NCHUNK + NBUF - 1) // NBUF) * NBUF


@functools.partial(
    pl.kernel,
    out_type=jax.ShapeDtypeStruct((NC, N_PAD, HALF), jnp.float32),
    mesh=_mesh,
    compiler_params=_sc_params,
    scratch_types=[
        pltpu.VMEM((EPT_B,), jnp.int32),
        pltpu.VMEM((CHUNK,), jnp.int32),
        pltpu.VMEM((CHUNK,), jnp.int32),
        pltpu.VMEM((CHUNK,), jnp.int32),
        pltpu.VMEM((CHUNK,), jnp.int32),
        pltpu.VMEM((CHUNK,), jnp.int32),
        pltpu.VMEM((CHUNK,), jnp.int32),
        pltpu.VMEM((NBUF, CHUNK, HALF), jnp.float32),
        pltpu.VMEM_SHARED((N_PAD, HALF), jnp.float32),
        pltpu.SemaphoreType.DMA,
        pltpu.SemaphoreType.DMA,
        pltpu.SemaphoreType.DMA,
    ],
)
def _segsum_kernel(g_hbm, pk_hbm, z_hbm, s_out,
                   pk_v, si0, si1, si2, di0, di1, di2, rows_v, acc,
                   gs0, gs1, gs2):
    c = lax.axis_index("c")
    s = lax.axis_index("s")
    srcs = [si0, si1, si2]
    dsts = [di0, di1, di2]
    gsems = [gs0, gs1, gs2]
    stripe = pl.ds(s * STRIPE, STRIPE)
    base = c * N

    pltpu.sync_copy(z_hbm, acc.at[stripe])
    pltpu.sync_copy(pk_hbm.at[s], pk_v)
    plsc.subcore_barrier()

    def unpack(j, b):
        @pl.loop(0, CHUNK // LANES)
        def _(k):
            w = pk_v[pl.ds(j * CHUNK + k * LANES, LANES)]
            srcs[b][pl.ds(k * LANES, LANES)] = (w & 0xFFFF) + base
            dsts[b][pl.ds(k * LANES, LANES)] = lax.shift_right_logical(w, 16)

    for b in range(NBUF):
        unpack(b, b)
        pltpu.async_copy(g_hbm.at[srcs[b]], rows_v.at[b], gsems[b])

    @pl.loop(0, _NITER, step=NBUF)
    def _(j0):
        for b in range(NBUF):
            j = j0 + b

            @pl.when(j < NCHUNK)
            def _():
                pltpu.make_async_copy(g_hbm.at[srcs[b]], rows_v.at[b],
                                      gsems[b]).wait()
                pltpu.sync_copy(rows_v.at[b], acc.at[dsts[b]], add=True)
                nxt = j + NBUF

                @pl.when(nxt < NCHUNK)
                def _():
                    unpack(nxt, b)
                    pltpu.async_copy(g_hbm.at[srcs[b]], rows_v.at[b], gsems[b])

    plsc.subcore_barrier()
    pltpu.sync_copy(acc.at[stripe], s_out.at[c, stripe])


def _prep_body(x_ref, g0_ref, b0_ref, d0_ref, d1_ref, h_ref, g_ref, dinv_ref):
    xv = x_ref[...]
    mean = jnp.mean(xv, axis=-1, keepdims=True)
    var = jnp.mean(jnp.square(xv - mean), axis=-1, keepdims=True)
    hv = (xv - mean) * lax.rsqrt(var + LN_EPS) * g0_ref[...] + b0_ref[...]
    hv = jnp.maximum(hv, 0.0)
    deg = d0_ref[...] + d1_ref[...] + 1.0
    dinv = lax.rsqrt(deg)
    gv = dinv * hv
    h_ref[...] = hv
    dinv_ref[...] = dinv
    g_ref[...] = jnp.stack([gv[:, :HALF], gv[:, HALF:]], axis=0)


def _combine_body(s_ref, h_ref, dinv_ref, w_ref, out_ref):
    s_all = jnp.concatenate([s_ref[0], s_ref[1]], axis=-1)
    dinv = dinv_ref[...]
    agg = dinv * s_all + (dinv * dinv) * h_ref[...]
    t = (1.0 - ALPHA) * agg + ALPHA * h_ref[...]
    mm = lax.dot(
        t, w_ref[...],
        precision=lax.Precision.HIGHEST,
        preferred_element_type=jnp.float32,
    )
    out_ref[...] = (1.0 - BETA) * t + BETA * mm


_ROWS_BLK = 1000
_GRID = N // _ROWS_BLK


def kernel(x, edge_index, ln_gamma, ln_beta, weight1):
    src = edge_index[0]
    dst = edge_index[1]
    npad = E_PAD - E
    dst_p = jnp.concatenate([dst, jnp.full((npad,), PAD_DST, jnp.int32)])

    dstA = dst_p.reshape(NC * NS, EPT_A)
    pk = ((dst << 16) | src).reshape(NS, EPT_B)

    degA = _deg_kernel(dstA)
    d0 = degA[0, :N].reshape(N, 1)
    d1 = degA[1, :N].reshape(N, 1)

    h, g_st, dinv = pl.pallas_call(
        _prep_body,
        grid=(_GRID,),
        in_specs=[
            pl.BlockSpec((_ROWS_BLK, C), lambda i: (i, 0)),
            pl.BlockSpec((1, C), lambda i: (0, 0)),
            pl.BlockSpec((1, C), lambda i: (0, 0)),
            pl.BlockSpec((_ROWS_BLK, 1), lambda i: (i, 0)),
            pl.BlockSpec((_ROWS_BLK, 1), lambda i: (i, 0)),
        ],
        out_specs=[
            pl.BlockSpec((_ROWS_BLK, C), lambda i: (i, 0)),
            pl.BlockSpec((NC, _ROWS_BLK, HALF), lambda i: (0, i, 0)),
            pl.BlockSpec((_ROWS_BLK, 1), lambda i: (i, 0)),
        ],
        out_shape=[
            jax.ShapeDtypeStruct((N, C), jnp.float32),
            jax.ShapeDtypeStruct((NC, N, HALF), jnp.float32),
            jax.ShapeDtypeStruct((N, 1), jnp.float32),
        ],
    )(x, ln_gamma.reshape(1, C), ln_beta.reshape(1, C), d0, d1)

    g_flat = g_st.reshape(NC * N, HALF)
    zrows = jnp.zeros((STRIPE, HALF), jnp.float32)
    s_sum = _segsum_kernel(g_flat, pk, zrows)

    out = pl.pallas_call(
        _combine_body,
        grid=(_GRID,),
        in_specs=[
            pl.BlockSpec((NC, _ROWS_BLK, HALF), lambda i: (0, i, 0)),
            pl.BlockSpec((_ROWS_BLK, C), lambda i: (i, 0)),
            pl.BlockSpec((_ROWS_BLK, 1), lambda i: (i, 0)),
            pl.BlockSpec((C, C), lambda i: (0, 0)),
        ],
        out_specs=pl.BlockSpec((_ROWS_BLK, C), lambda i: (i, 0)),
        out_shape=jax.ShapeDtypeStruct((N, C), jnp.float32),
    )(s_sum, h, dinv, weight1)
    return out
